# baseline (speedup 1.0000x reference)
import jax
import jax.numpy as jnp
from jax.experimental import pallas as pl
from jax.experimental.pallas import tpu as pltpu

_BLOCK_ROWS = 2000


def _tc_body(id_ref, table_ref, x_ref, o_ref):
    ct = id_ref[0]
    row = table_ref[pl.ds(ct, 1), :]
    o_ref[...] = x_ref[...] + row


def kernel(x, cell_type_id, table):
    n, d = x.shape
    x2 = x.reshape(n // 2, 2 * d)
    table2 = jnp.concatenate([table, table], axis=1)
    ct = cell_type_id.astype(jnp.int32)
    rows = n // 2
    grid = rows // _BLOCK_ROWS

    out2 = pl.pallas_call(
        _tc_body,
        grid=(grid,),
        in_specs=[
            pl.BlockSpec(memory_space=pltpu.SMEM),
            pl.BlockSpec((table2.shape[0], 2 * d), lambda i: (0, 0)),
            pl.BlockSpec((_BLOCK_ROWS, 2 * d), lambda i: (i, 0)),
        ],
        out_specs=pl.BlockSpec((_BLOCK_ROWS, 2 * d), lambda i: (i, 0)),
        out_shape=jax.ShapeDtypeStruct((rows, 2 * d), jnp.float32),
        compiler_params=pltpu.CompilerParams(
            dimension_semantics=("parallel",),
        ),
    )(ct, table2, x2)
    return out2.reshape(n, d)


# speedup vs baseline: 1.5017x; 1.5017x over previous
import jax
import jax.numpy as jnp
from jax.experimental import pallas as pl
from jax.experimental.pallas import tpu as pltpu

_BLOCK_ROWS = 4000


def _tc_body(id_ref, table_ref, x_ref, o_ref):
    ct = id_ref[0]
    row = table_ref[pl.ds(ct, 1), :]
    o_ref[...] = x_ref[...] + row


def kernel(x, cell_type_id, table):
    n, d = x.shape
    ct = cell_type_id.astype(jnp.int32)
    grid = n // _BLOCK_ROWS

    return pl.pallas_call(
        _tc_body,
        grid=(grid,),
        in_specs=[
            pl.BlockSpec(memory_space=pltpu.SMEM),
            pl.BlockSpec((table.shape[0], d), lambda i: (0, 0)),
            pl.BlockSpec((_BLOCK_ROWS, d), lambda i: (i, 0)),
        ],
        out_specs=pl.BlockSpec((_BLOCK_ROWS, d), lambda i: (i, 0)),
        out_shape=jax.ShapeDtypeStruct((n, d), jnp.float32),
        compiler_params=pltpu.CompilerParams(
            dimension_semantics=("parallel",),
        ),
    )(ct, table, x)


# speedup vs baseline: 1.5529x; 1.0341x over previous
import jax
import jax.numpy as jnp
from jax.experimental import pallas as pl
from jax.experimental.pallas import tpu as pltpu

_BLOCK_ROWS = 20000


def _tc_body(id_ref, table_ref, x_ref, o_ref):
    ct = id_ref[0]
    row = table_ref[pl.ds(ct, 1), :]
    o_ref[...] = x_ref[...] + row


def kernel(x, cell_type_id, table):
    n, d = x.shape
    ct = cell_type_id.astype(jnp.int32)
    grid = n // _BLOCK_ROWS

    return pl.pallas_call(
        _tc_body,
        grid=(grid,),
        in_specs=[
            pl.BlockSpec(memory_space=pltpu.SMEM),
            pl.BlockSpec((table.shape[0], d), lambda i: (0, 0)),
            pl.BlockSpec((_BLOCK_ROWS, d), lambda i: (i, 0)),
        ],
        out_specs=pl.BlockSpec((_BLOCK_ROWS, d), lambda i: (i, 0)),
        out_shape=jax.ShapeDtypeStruct((n, d), jnp.float32),
        compiler_params=pltpu.CompilerParams(
            dimension_semantics=("parallel",),
        ),
    )(ct, table, x)


# speedup vs baseline: 5.8199x; 3.7478x over previous
import jax
import jax.numpy as jnp
from jax.experimental import pallas as pl
from jax.experimental.pallas import tpu as pltpu

_BLOCK_COLS = 4096


def _tc_body(id_ref, tt_ref, x_ref, o_ref):
    ct = id_ref[0]
    tt = tt_ref[...]
    lane = jax.lax.broadcasted_iota(jnp.int32, tt.shape, 1)
    col = jnp.sum(jnp.where(lane == ct, tt, 0.0), axis=1, keepdims=True)
    o_ref[...] = x_ref[...] + col


def kernel(x, cell_type_id, table):
    n, d = x.shape
    xt = x.T
    tt = table.T
    ct = cell_type_id.astype(jnp.int32)
    grid = pl.cdiv(n, _BLOCK_COLS)

    outt = pl.pallas_call(
        _tc_body,
        grid=(grid,),
        in_specs=[
            pl.BlockSpec(memory_space=pltpu.SMEM),
            pl.BlockSpec((d, tt.shape[1]), lambda i: (0, 0)),
            pl.BlockSpec((d, _BLOCK_COLS), lambda i: (0, i)),
        ],
        out_specs=pl.BlockSpec((d, _BLOCK_COLS), lambda i: (0, i)),
        out_shape=jax.ShapeDtypeStruct((d, n), jnp.float32),
        compiler_params=pltpu.CompilerParams(
            dimension_semantics=("parallel",),
        ),
    )(ct, tt, xt)
    return outt.T


# speedup vs baseline: 8.9980x; 1.5461x over previous
import jax
import jax.numpy as jnp
from jax.experimental import pallas as pl
from jax.experimental.pallas import tpu as pltpu

_BLOCK_COLS = 16384


def _tc_body(id_ref, tt_ref, x_ref, o_ref):
    ct = id_ref[0]
    tt = tt_ref[...]
    lane = jax.lax.broadcasted_iota(jnp.int32, tt.shape, 1)
    col = jnp.sum(jnp.where(lane == ct, tt, 0.0), axis=1, keepdims=True)
    o_ref[...] = x_ref[...] + col


def kernel(x, cell_type_id, table):
    n, d = x.shape
    xt = x.T
    tt = table.T
    ct = cell_type_id.astype(jnp.int32)
    grid = pl.cdiv(n, _BLOCK_COLS)

    outt = pl.pallas_call(
        _tc_body,
        grid=(grid,),
        in_specs=[
            pl.BlockSpec(memory_space=pltpu.SMEM),
            pl.BlockSpec((d, tt.shape[1]), lambda i: (0, 0)),
            pl.BlockSpec((d, _BLOCK_COLS), lambda i: (0, i)),
        ],
        out_specs=pl.BlockSpec((d, _BLOCK_COLS), lambda i: (0, i)),
        out_shape=jax.ShapeDtypeStruct((d, n), jnp.float32),
        compiler_params=pltpu.CompilerParams(
            dimension_semantics=("parallel",),
        ),
    )(ct, tt, xt)
    return outt.T


# speedup vs baseline: 9.4072x; 1.0455x over previous
import jax
import jax.numpy as jnp
from jax.experimental import pallas as pl
from jax.experimental.pallas import tpu as pltpu

_BLOCK_COLS = 32768


def _tc_body(id_ref, tt_ref, x_ref, o_ref):
    ct = id_ref[0]
    tt = tt_ref[...]
    lane = jax.lax.broadcasted_iota(jnp.int32, tt.shape, 1)
    col = jnp.sum(jnp.where(lane == ct, tt, 0.0), axis=1, keepdims=True)
    o_ref[...] = x_ref[...] + col


def kernel(x, cell_type_id, table):
    n, d = x.shape
    xt = x.T
    tt = table.T
    ct = cell_type_id.astype(jnp.int32)
    grid = pl.cdiv(n, _BLOCK_COLS)

    outt = pl.pallas_call(
        _tc_body,
        grid=(grid,),
        in_specs=[
            pl.BlockSpec(memory_space=pltpu.SMEM),
            pl.BlockSpec((d, tt.shape[1]), lambda i: (0, 0)),
            pl.BlockSpec((d, _BLOCK_COLS), lambda i: (0, i)),
        ],
        out_specs=pl.BlockSpec((d, _BLOCK_COLS), lambda i: (0, i)),
        out_shape=jax.ShapeDtypeStruct((d, n), jnp.float32),
        compiler_params=pltpu.CompilerParams(
            dimension_semantics=("parallel",),
        ),
    )(ct, tt, xt)
    return outt.T
